# Initial kernel scaffold; baseline (speedup 1.0000x reference)
import jax
import jax.numpy as jnp
from jax.experimental import pallas as pl


def kernel(t, batch_question, batch_sent_len, batch_path_hidden, last_r, r_space, e_space, action_mask, word_emb, rel_emb, ent_emb, W_sa, b_sa, W_att, b_att, W1, b1, W2, b2):
    raise NotImplementedError("write your pallas kernel here")


# speedup vs baseline: 200.1218x; 200.1218x over previous
import functools

import jax
import jax.numpy as jnp
from jax import lax
from jax.experimental import pallas as pl
from jax.experimental.pallas import tpu as pltpu
from jax.experimental.pallas import tpu_sc as plsc

B, S, R, A = 32, 64, 128, 256
WORD_DIM = 128
REL_DIM = 128
ENT_DIM = 128
HIST_DIM = 256
MAX_HOP = 3
ACTION_DIM = REL_DIM + ENT_DIM
NO_OP = 2
NEG = -1e9


def _sc_gather(qidx, eidx, word_emb, ent_emb):
    info = plsc.get_sparse_core_info()
    nc, ns = info.num_cores, info.num_subcores
    nw = nc * ns
    qn = qidx.shape[0]
    en = eidx.shape[0]
    qpw = qn // nw
    epw = en // nw
    ech = min(epw, 128)
    n_ech = epw // ech
    mesh = plsc.VectorSubcoreMesh(core_axis_name="c", subcore_axis_name="s")

    @functools.partial(
        pl.kernel,
        out_type=(
            jax.ShapeDtypeStruct((qn, WORD_DIM), jnp.float32),
            jax.ShapeDtypeStruct((en, ENT_DIM), jnp.float32),
        ),
        mesh=mesh,
        scratch_types=[
            pltpu.VMEM((qpw,), jnp.int32),
            pltpu.VMEM((qpw, WORD_DIM), jnp.float32),
            pltpu.VMEM((ech,), jnp.int32),
            pltpu.VMEM((ech, ENT_DIM), jnp.float32),
            pltpu.SemaphoreType.DMA,
        ],
    )
    def k(qidx_hbm, eidx_hbm, word_hbm, ent_hbm, qout_hbm, eout_hbm,
          qi_v, qr_v, ei_v, er_v, sem):
        wid = lax.axis_index("s") * nc + lax.axis_index("c")
        qb = wid * qpw
        pltpu.sync_copy(qidx_hbm.at[pl.ds(qb, qpw)], qi_v)
        pltpu.async_copy(word_hbm.at[qi_v], qr_v, sem).wait()
        pltpu.sync_copy(qr_v, qout_hbm.at[pl.ds(qb, qpw)])
        for j in range(n_ech):
            eb = wid * epw + j * ech
            pltpu.sync_copy(eidx_hbm.at[pl.ds(eb, ech)], ei_v)
            pltpu.async_copy(ent_hbm.at[ei_v], er_v, sem).wait()
            pltpu.sync_copy(er_v, eout_hbm.at[pl.ds(eb, ech)])

    return k(qidx, eidx, word_emb, ent_emb)


def _tc_body(tt_ref, slen_ref, lastr_ref, batt_ref,
             qe_ref, wsa_ref, bsa_ref, watt_ref, rel_ref, ph_ref,
             w1_ref, b1_ref, w2_ref, b2_ref, er_ref, rsp_ref, am_ref,
             out_ref):
    f32 = jnp.float32
    b = pl.program_id(0)
    sv = jnp.tanh(
        jnp.dot(qe_ref[0], wsa_ref[...], preferred_element_type=f32)
        + bsa_ref[...])
    u = sv * watt_ref[...]
    logits = lax.dot_general(u, rel_ref[...], (((1,), (1,)), ((), ())),
                             preferred_element_type=f32) + batt_ref[0]
    sids = lax.broadcasted_iota(jnp.int32, (S, 1), 0)
    logits = jnp.where(sids >= slen_ref[b], NEG, logits)
    m = jnp.max(logits, axis=0, keepdims=True)
    e = jnp.exp(logits - m)
    alpha = e / jnp.sum(e, axis=0, keepdims=True)
    raq = lax.dot_general(alpha, sv, (((0,), (0,)), ((), ())),
                          preferred_element_type=f32)
    w1 = w1_ref[...]
    base = jnp.dot(ph_ref[0], w1[:HIST_DIM],
                   preferred_element_type=f32)
    y = jnp.dot(raq, w1[HIST_DIM:], preferred_element_type=f32)
    z = jnp.maximum(base + y + b1_ref[...], 0.0)
    x2 = jnp.dot(z, w2_ref[...], preferred_element_type=f32) + b2_ref[...]
    rsp = rsp_ref[0]
    oht = (rsp == lax.broadcasted_iota(jnp.int32, (R, A), 0)).astype(f32)
    g = lax.dot_general(oht, x2, (((0,), (0,)), ((), ())),
                        preferred_element_type=f32)
    relg = lax.dot_general(oht, rel_ref[...], (((0,), (0,)), ((), ())),
                           preferred_element_type=f32)
    aemb = jnp.concatenate([relg, er_ref[0]], axis=1)
    scores = jnp.sum(g * aemb, axis=1)[None, :]
    amask = am_ref[0]
    is_noop = (rsp == NO_OP).astype(jnp.int32)
    am_first = (1 - is_noop) * amask
    am_last = is_noop * amask
    jm = 1 - (lastr_ref[b] == NO_OP).astype(jnp.int32)
    aids = lax.broadcasted_iota(jnp.int32, (1, A), 1)
    selfl = (aids == 0).astype(jnp.int32)
    am_mid = jm * amask + (1 - jm) * selfl
    tt = tt_ref[0]
    am = jnp.where(tt == 0, am_first,
                   jnp.where(tt == MAX_HOP - 1, am_last, am_mid))
    scores = jnp.where(am > 0, scores, NEG)
    mm = jnp.max(scores, axis=1, keepdims=True)
    ee = jnp.exp(scores - mm)
    out_ref[0] = ee / jnp.sum(ee, axis=1, keepdims=True)


def _tc_call_kwargs():
    smem = pl.BlockSpec(memory_space=pltpu.SMEM)
    full = pl.BlockSpec
    return dict(
        grid=(B,),
        in_specs=[
            smem,
            smem,
            smem,
            smem,
            full((1, S, WORD_DIM), lambda b: (b, 0, 0)),
            full((WORD_DIM, REL_DIM), lambda b: (0, 0)),
            full((1, REL_DIM), lambda b: (0, 0)),
            full((1, REL_DIM), lambda b: (0, 0)),
            full((R, REL_DIM), lambda b: (0, 0)),
            full((1, 1, HIST_DIM), lambda b: (b, 0, 0)),
            full((HIST_DIM + REL_DIM, ACTION_DIM), lambda b: (0, 0)),
            full((1, ACTION_DIM), lambda b: (0, 0)),
            full((ACTION_DIM, ACTION_DIM), lambda b: (0, 0)),
            full((1, ACTION_DIM), lambda b: (0, 0)),
            full((1, A, ENT_DIM), lambda b: (b, 0, 0)),
            full((1, 1, A), lambda b: (b, 0, 0)),
            full((1, 1, A), lambda b: (b, 0, 0)),
        ],
        out_specs=full((1, 1, A), lambda b: (b, 0, 0)),
        out_shape=jax.ShapeDtypeStruct((B, 1, A), jnp.float32),
        compiler_params=pltpu.CompilerParams(
            dimension_semantics=("arbitrary",)),
    )


def kernel(t, batch_question, batch_sent_len, batch_path_hidden, last_r,
           r_space, e_space, action_mask, word_emb, rel_emb, ent_emb,
           W_sa, b_sa, W_att, b_att, W1, b1, W2, b2):
    qidx = batch_question.reshape(B * S).astype(jnp.int32)
    eidx = e_space.reshape(B * A).astype(jnp.int32)
    q_rows, e_rows = _sc_gather(qidx, eidx, word_emb, ent_emb)

    tt = jnp.asarray(t, jnp.int32).reshape(1)
    wsa_t = jnp.take(W_sa, t, axis=0)
    bsa_t = jnp.take(b_sa, t, axis=0).reshape(1, REL_DIM)
    watt_row = W_att[:, 0].reshape(1, REL_DIM)

    probs = pl.pallas_call(_tc_body, **_tc_call_kwargs())(
        tt,
        batch_sent_len.astype(jnp.int32),
        last_r.astype(jnp.int32),
        b_att,
        q_rows.reshape(B, S, WORD_DIM),
        wsa_t,
        bsa_t,
        watt_row,
        rel_emb,
        batch_path_hidden.reshape(B, 1, HIST_DIM),
        W1,
        b1.reshape(1, ACTION_DIM),
        W2,
        b2.reshape(1, ACTION_DIM),
        e_rows.reshape(B, A, ENT_DIM),
        r_space.reshape(B, 1, A).astype(jnp.int32),
        action_mask.reshape(B, 1, A).astype(jnp.int32),
    )
    return probs.reshape(B, A)


# speedup vs baseline: 227.7070x; 1.1378x over previous
import functools

import jax
import jax.numpy as jnp
from jax import lax
from jax.experimental import pallas as pl
from jax.experimental.pallas import tpu as pltpu
from jax.experimental.pallas import tpu_sc as plsc

B, S, R, A = 32, 64, 128, 256
WORD_DIM = 128
REL_DIM = 128
ENT_DIM = 128
HIST_DIM = 256
MAX_HOP = 3
ACTION_DIM = REL_DIM + ENT_DIM
NO_OP = 2
NEG = -1e9


def _sc_gather(qidx, eidx, word_emb, ent_emb):
    info = plsc.get_sparse_core_info()
    nc, ns = info.num_cores, info.num_subcores
    nw = nc * ns
    qn = qidx.shape[0]
    en = eidx.shape[0]
    qpw = qn // nw
    epw = en // nw
    ech = min(epw, 128)
    n_ech = epw // ech
    mesh = plsc.VectorSubcoreMesh(core_axis_name="c", subcore_axis_name="s")

    @functools.partial(
        pl.kernel,
        out_type=(
            jax.ShapeDtypeStruct((qn, WORD_DIM), jnp.float32),
            jax.ShapeDtypeStruct((en, ENT_DIM), jnp.float32),
        ),
        mesh=mesh,
        scratch_types=[
            pltpu.VMEM((qpw,), jnp.int32),
            pltpu.VMEM((qpw, WORD_DIM), jnp.float32),
            pltpu.VMEM((ech,), jnp.int32),
            pltpu.VMEM((ech, ENT_DIM), jnp.float32),
            pltpu.SemaphoreType.DMA,
        ],
    )
    def k(qidx_hbm, eidx_hbm, word_hbm, ent_hbm, qout_hbm, eout_hbm,
          qi_v, qr_v, ei_v, er_v, sem):
        wid = lax.axis_index("s") * nc + lax.axis_index("c")
        qb = wid * qpw
        pltpu.sync_copy(qidx_hbm.at[pl.ds(qb, qpw)], qi_v)
        pltpu.async_copy(word_hbm.at[qi_v], qr_v, sem).wait()
        pltpu.sync_copy(qr_v, qout_hbm.at[pl.ds(qb, qpw)])
        for j in range(n_ech):
            eb = wid * epw + j * ech
            pltpu.sync_copy(eidx_hbm.at[pl.ds(eb, ech)], ei_v)
            pltpu.async_copy(ent_hbm.at[ei_v], er_v, sem).wait()
            pltpu.sync_copy(er_v, eout_hbm.at[pl.ds(eb, ech)])

    return k(qidx, eidx, word_emb, ent_emb)


NB = 4


def _tc_body(tt_ref, slen_ref, lastr_ref, batt_ref,
             qe_ref, wsa_ref, bsa_ref, watt_ref, rel_ref, ph_ref,
             w1_ref, b1_ref, w2_ref, b2_ref, er_ref, rsp_ref, am_ref,
             out_ref):
    f32 = jnp.float32
    g0 = pl.program_id(0)
    wsa = wsa_ref[...]
    bsa = bsa_ref[...]
    watt = watt_ref[...]
    rel = rel_ref[...]
    w1 = w1_ref[...]
    w2 = w2_ref[...]
    b1v = b1_ref[...]
    b2v = b2_ref[...]
    tt = tt_ref[0]
    aids = lax.broadcasted_iota(jnp.int32, (1, A), 1)
    sids = lax.broadcasted_iota(jnp.int32, (S, 1), 0)
    riota = lax.broadcasted_iota(jnp.int32, (R, A), 0)
    for i in range(NB):
        b = g0 * NB + i
        sv = jnp.tanh(
            jnp.dot(qe_ref[i], wsa, preferred_element_type=f32) + bsa)
        u = sv * watt
        logits = lax.dot_general(u, rel, (((1,), (1,)), ((), ())),
                                 preferred_element_type=f32) + batt_ref[0]
        logits = jnp.where(sids >= slen_ref[b], NEG, logits)
        m = jnp.max(logits, axis=0, keepdims=True)
        e = jnp.exp(logits - m)
        alpha = e / jnp.sum(e, axis=0, keepdims=True)
        raq = lax.dot_general(alpha, sv, (((0,), (0,)), ((), ())),
                              preferred_element_type=f32)
        base = jnp.dot(ph_ref[i], w1[:HIST_DIM],
                       preferred_element_type=f32)
        y = jnp.dot(raq, w1[HIST_DIM:], preferred_element_type=f32)
        z = jnp.maximum(base + y + b1v, 0.0)
        x2 = jnp.dot(z, w2, preferred_element_type=f32) + b2v
        s_rel = jnp.sum(x2[:, :REL_DIM] * rel, axis=1, keepdims=True)
        rsp = rsp_ref[i]
        oht = (rsp == riota).astype(f32)
        gent = lax.dot_general(oht, x2[:, REL_DIM:],
                               (((0,), (0,)), ((), ())),
                               preferred_element_type=f32)
        srelg = lax.dot_general(s_rel, oht, (((0,), (0,)), ((), ())),
                                preferred_element_type=f32)
        scores = jnp.sum(gent * er_ref[i], axis=1)[None, :] + srelg
        amask = am_ref[i]
        is_noop = (rsp == NO_OP).astype(jnp.int32)
        am_first = (1 - is_noop) * amask
        am_last = is_noop * amask
        jm = 1 - (lastr_ref[b] == NO_OP).astype(jnp.int32)
        selfl = (aids == 0).astype(jnp.int32)
        am_mid = jm * amask + (1 - jm) * selfl
        am = jnp.where(tt == 0, am_first,
                       jnp.where(tt == MAX_HOP - 1, am_last, am_mid))
        scores = jnp.where(am > 0, scores, NEG)
        mm = jnp.max(scores, axis=1, keepdims=True)
        ee = jnp.exp(scores - mm)
        out_ref[i] = ee / jnp.sum(ee, axis=1, keepdims=True)


def _tc_call_kwargs():
    smem = pl.BlockSpec(memory_space=pltpu.SMEM)
    full = pl.BlockSpec
    return dict(
        grid=(B // NB,),
        in_specs=[
            smem,
            smem,
            smem,
            smem,
            full((NB, S, WORD_DIM), lambda b: (b, 0, 0)),
            full((WORD_DIM, REL_DIM), lambda b: (0, 0)),
            full((1, REL_DIM), lambda b: (0, 0)),
            full((1, REL_DIM), lambda b: (0, 0)),
            full((R, REL_DIM), lambda b: (0, 0)),
            full((NB, 1, HIST_DIM), lambda b: (b, 0, 0)),
            full((HIST_DIM + REL_DIM, ACTION_DIM), lambda b: (0, 0)),
            full((1, ACTION_DIM), lambda b: (0, 0)),
            full((ACTION_DIM, ACTION_DIM), lambda b: (0, 0)),
            full((1, ACTION_DIM), lambda b: (0, 0)),
            full((NB, A, ENT_DIM), lambda b: (b, 0, 0)),
            full((NB, 1, A), lambda b: (b, 0, 0)),
            full((NB, 1, A), lambda b: (b, 0, 0)),
        ],
        out_specs=full((NB, 1, A), lambda b: (b, 0, 0)),
        out_shape=jax.ShapeDtypeStruct((B, 1, A), jnp.float32),
        compiler_params=pltpu.CompilerParams(
            dimension_semantics=("arbitrary",)),
    )


def kernel(t, batch_question, batch_sent_len, batch_path_hidden, last_r,
           r_space, e_space, action_mask, word_emb, rel_emb, ent_emb,
           W_sa, b_sa, W_att, b_att, W1, b1, W2, b2):
    qidx = batch_question.reshape(B * S).astype(jnp.int32)
    eidx = e_space.reshape(B * A).astype(jnp.int32)
    q_rows, e_rows = _sc_gather(qidx, eidx, word_emb, ent_emb)

    tt = jnp.asarray(t, jnp.int32).reshape(1)
    wsa_t = jnp.take(W_sa, t, axis=0)
    bsa_t = jnp.take(b_sa, t, axis=0).reshape(1, REL_DIM)
    watt_row = W_att[:, 0].reshape(1, REL_DIM)

    probs = pl.pallas_call(_tc_body, **_tc_call_kwargs())(
        tt,
        batch_sent_len.astype(jnp.int32),
        last_r.astype(jnp.int32),
        b_att,
        q_rows.reshape(B, S, WORD_DIM),
        wsa_t,
        bsa_t,
        watt_row,
        rel_emb,
        batch_path_hidden.reshape(B, 1, HIST_DIM),
        W1,
        b1.reshape(1, ACTION_DIM),
        W2,
        b2.reshape(1, ACTION_DIM),
        e_rows.reshape(B, A, ENT_DIM),
        r_space.reshape(B, 1, A).astype(jnp.int32),
        action_mask.reshape(B, 1, A).astype(jnp.int32),
    )
    return probs.reshape(B, A)


# speedup vs baseline: 231.8890x; 1.0184x over previous
import functools

import jax
import jax.numpy as jnp
from jax import lax
from jax.experimental import pallas as pl
from jax.experimental.pallas import tpu as pltpu
from jax.experimental.pallas import tpu_sc as plsc

B, S, R, A = 32, 64, 128, 256
WORD_DIM = 128
REL_DIM = 128
ENT_DIM = 128
HIST_DIM = 256
MAX_HOP = 3
ACTION_DIM = REL_DIM + ENT_DIM
NO_OP = 2
NEG = -1e9


def _sc_gather(qidx, eidx, word_emb, ent_emb):
    info = plsc.get_sparse_core_info()
    nc, ns = info.num_cores, info.num_subcores
    nw = nc * ns
    qn = qidx.shape[0]
    en = eidx.shape[0]
    qpw = qn // nw
    epw = en // nw
    ech = min(epw, 128)
    n_ech = epw // ech
    mesh = plsc.VectorSubcoreMesh(core_axis_name="c", subcore_axis_name="s")

    @functools.partial(
        pl.kernel,
        out_type=(
            jax.ShapeDtypeStruct((qn, WORD_DIM), jnp.float32),
            jax.ShapeDtypeStruct((en, ENT_DIM), jnp.float32),
        ),
        mesh=mesh,
        scratch_types=[
            pltpu.VMEM((qpw,), jnp.int32),
            pltpu.VMEM((qpw, WORD_DIM), jnp.float32),
            pltpu.VMEM((ech,), jnp.int32),
            pltpu.VMEM((ech, ENT_DIM), jnp.float32),
            pltpu.SemaphoreType.DMA,
        ],
    )
    def k(qidx_hbm, eidx_hbm, word_hbm, ent_hbm, qout_hbm, eout_hbm,
          qi_v, qr_v, ei_v, er_v, sem):
        wid = lax.axis_index("s") * nc + lax.axis_index("c")
        qb = wid * qpw
        pltpu.sync_copy(qidx_hbm.at[pl.ds(qb, qpw)], qi_v)
        pltpu.async_copy(word_hbm.at[qi_v], qr_v, sem).wait()
        pltpu.sync_copy(qr_v, qout_hbm.at[pl.ds(qb, qpw)])
        for j in range(n_ech):
            eb = wid * epw + j * ech
            pltpu.sync_copy(eidx_hbm.at[pl.ds(eb, ech)], ei_v)
            pltpu.async_copy(ent_hbm.at[ei_v], er_v, sem).wait()
            pltpu.sync_copy(er_v, eout_hbm.at[pl.ds(eb, ech)])

    return k(qidx, eidx, word_emb, ent_emb)


NB = 4


def _tc_body(tt_ref, slen_ref, lastr_ref, batt_ref,
             qe_ref, wsa_ref, bsa_ref, watt_ref, rel_ref, ph_ref,
             w1_ref, b1_ref, w2_ref, b2_ref, er_ref, rsp_ref, am_ref,
             out_ref):
    f32 = jnp.float32
    g0 = pl.program_id(0)
    wsa = wsa_ref[...]
    bsa = bsa_ref[...]
    watt = watt_ref[...]
    rel = rel_ref[...]
    w1 = w1_ref[...]
    w2 = w2_ref[...]
    b1v = b1_ref[...]
    b2v = b2_ref[...]
    tt = tt_ref[0]
    aids = lax.broadcasted_iota(jnp.int32, (1, A), 1)
    sids = lax.broadcasted_iota(jnp.int32, (S, 1), 0)
    riota = lax.broadcasted_iota(jnp.int32, (R, A), 0)
    ones_col = jnp.ones((REL_DIM, 1), f32)
    ones_row = jnp.ones((1, R), f32)
    for i in range(NB):
        b = g0 * NB + i
        sv = jnp.tanh(
            jnp.dot(qe_ref[i], wsa, preferred_element_type=f32) + bsa)
        u = sv * watt
        logits = lax.dot_general(u, rel, (((1,), (1,)), ((), ())),
                                 preferred_element_type=f32) + batt_ref[0]
        logits = jnp.where(sids >= slen_ref[b], NEG, logits)
        m = jnp.max(logits, axis=0, keepdims=True)
        e = jnp.exp(logits - m)
        alpha = e / jnp.sum(e, axis=0, keepdims=True)
        raq = lax.dot_general(alpha, sv, (((0,), (0,)), ((), ())),
                              preferred_element_type=f32)
        base = jnp.dot(ph_ref[i], w1[:HIST_DIM],
                       preferred_element_type=f32)
        y = jnp.dot(raq, w1[HIST_DIM:], preferred_element_type=f32)
        z = jnp.maximum(base + y + b1v, 0.0)
        x2 = jnp.dot(z, w2, preferred_element_type=f32) + b2v
        s_rel = jnp.dot(x2[:, :REL_DIM] * rel, ones_col,
                        preferred_element_type=f32)
        p = lax.dot_general(x2[:, REL_DIM:], er_ref[i],
                            (((1,), (1,)), ((), ())),
                            preferred_element_type=f32)
        rsp = rsp_ref[i]
        oht = (rsp == riota).astype(f32)
        scores = jnp.dot(ones_row, oht * (p + s_rel),
                         preferred_element_type=f32)
        amask = am_ref[i]
        is_noop = (rsp == NO_OP).astype(jnp.int32)
        am_first = (1 - is_noop) * amask
        am_last = is_noop * amask
        jm = 1 - (lastr_ref[b] == NO_OP).astype(jnp.int32)
        selfl = (aids == 0).astype(jnp.int32)
        am_mid = jm * amask + (1 - jm) * selfl
        am = jnp.where(tt == 0, am_first,
                       jnp.where(tt == MAX_HOP - 1, am_last, am_mid))
        scores = jnp.where(am > 0, scores, NEG)
        mm = jnp.max(scores, axis=1, keepdims=True)
        ee = jnp.exp(scores - mm)
        out_ref[i] = ee / jnp.sum(ee, axis=1, keepdims=True)


def _tc_call_kwargs():
    smem = pl.BlockSpec(memory_space=pltpu.SMEM)
    full = pl.BlockSpec
    return dict(
        grid=(B // NB,),
        in_specs=[
            smem,
            smem,
            smem,
            smem,
            full((NB, S, WORD_DIM), lambda b: (b, 0, 0)),
            full((WORD_DIM, REL_DIM), lambda b: (0, 0)),
            full((1, REL_DIM), lambda b: (0, 0)),
            full((1, REL_DIM), lambda b: (0, 0)),
            full((R, REL_DIM), lambda b: (0, 0)),
            full((NB, 1, HIST_DIM), lambda b: (b, 0, 0)),
            full((HIST_DIM + REL_DIM, ACTION_DIM), lambda b: (0, 0)),
            full((1, ACTION_DIM), lambda b: (0, 0)),
            full((ACTION_DIM, ACTION_DIM), lambda b: (0, 0)),
            full((1, ACTION_DIM), lambda b: (0, 0)),
            full((NB, A, ENT_DIM), lambda b: (b, 0, 0)),
            full((NB, 1, A), lambda b: (b, 0, 0)),
            full((NB, 1, A), lambda b: (b, 0, 0)),
        ],
        out_specs=full((NB, 1, A), lambda b: (b, 0, 0)),
        out_shape=jax.ShapeDtypeStruct((B, 1, A), jnp.float32),
        compiler_params=pltpu.CompilerParams(
            dimension_semantics=("arbitrary",)),
    )


def kernel(t, batch_question, batch_sent_len, batch_path_hidden, last_r,
           r_space, e_space, action_mask, word_emb, rel_emb, ent_emb,
           W_sa, b_sa, W_att, b_att, W1, b1, W2, b2):
    qidx = batch_question.reshape(B * S).astype(jnp.int32)
    eidx = e_space.reshape(B * A).astype(jnp.int32)
    q_rows, e_rows = _sc_gather(qidx, eidx, word_emb, ent_emb)

    tt = jnp.asarray(t, jnp.int32).reshape(1)
    wsa_t = jnp.take(W_sa, t, axis=0)
    bsa_t = jnp.take(b_sa, t, axis=0).reshape(1, REL_DIM)
    watt_row = W_att[:, 0].reshape(1, REL_DIM)

    probs = pl.pallas_call(_tc_body, **_tc_call_kwargs())(
        tt,
        batch_sent_len.astype(jnp.int32),
        last_r.astype(jnp.int32),
        b_att,
        q_rows.reshape(B, S, WORD_DIM),
        wsa_t,
        bsa_t,
        watt_row,
        rel_emb,
        batch_path_hidden.reshape(B, 1, HIST_DIM),
        W1,
        b1.reshape(1, ACTION_DIM),
        W2,
        b2.reshape(1, ACTION_DIM),
        e_rows.reshape(B, A, ENT_DIM),
        r_space.reshape(B, 1, A).astype(jnp.int32),
        action_mask.reshape(B, 1, A).astype(jnp.int32),
    )
    return probs.reshape(B, A)


# speedup vs baseline: 248.5469x; 1.0718x over previous
import functools

import jax
import jax.numpy as jnp
from jax import lax
from jax.experimental import pallas as pl
from jax.experimental.pallas import tpu as pltpu
from jax.experimental.pallas import tpu_sc as plsc

B, S, R, A = 32, 64, 128, 256
WORD_DIM = 128
REL_DIM = 128
ENT_DIM = 128
HIST_DIM = 256
MAX_HOP = 3
ACTION_DIM = REL_DIM + ENT_DIM
NO_OP = 2
NEG = -1e9

ECH = 128


def _sc_gather(batch_question, e_space, word_emb, ent_emb):
    info = plsc.get_sparse_core_info()
    nc, ns = info.num_cores, info.num_subcores
    mesh = plsc.VectorSubcoreMesh(core_axis_name="c", subcore_axis_name="s")

    @functools.partial(
        pl.kernel,
        out_type=(
            jax.ShapeDtypeStruct((B, S, WORD_DIM), jnp.float32),
            jax.ShapeDtypeStruct((B, A, ENT_DIM), jnp.float32),
        ),
        mesh=mesh,
        scratch_types=[
            pltpu.VMEM((S,), jnp.int32),
            pltpu.VMEM((ECH,), jnp.int32),
            pltpu.VMEM((ECH,), jnp.int32),
            pltpu.VMEM((S, WORD_DIM), jnp.float32),
            pltpu.VMEM((ECH, ENT_DIM), jnp.float32),
            pltpu.VMEM((ECH, ENT_DIM), jnp.float32),
            pltpu.SemaphoreType.DMA,
            pltpu.SemaphoreType.DMA,
            pltpu.SemaphoreType.DMA,
        ],
    )
    def k(qidx_hbm, eidx_hbm, word_hbm, ent_hbm, qout_hbm, eout_hbm,
          qi_v, ei0_v, ei1_v, qr_v, er0_v, er1_v, sem_q, sem_e0, sem_e1):
        wid = lax.axis_index("s") * nc + lax.axis_index("c")
        pltpu.sync_copy(qidx_hbm.at[wid], qi_v)
        pltpu.sync_copy(eidx_hbm.at[wid, pl.ds(0, ECH)], ei0_v)
        pltpu.sync_copy(eidx_hbm.at[wid, pl.ds(ECH, ECH)], ei1_v)
        cq = pltpu.async_copy(word_hbm.at[qi_v], qr_v, sem_q)
        c0 = pltpu.async_copy(ent_hbm.at[ei0_v], er0_v, sem_e0)
        c1 = pltpu.async_copy(ent_hbm.at[ei1_v], er1_v, sem_e1)
        cq.wait()
        pltpu.sync_copy(qr_v, qout_hbm.at[wid])
        c0.wait()
        pltpu.sync_copy(er0_v, eout_hbm.at[wid, pl.ds(0, ECH)])
        c1.wait()
        pltpu.sync_copy(er1_v, eout_hbm.at[wid, pl.ds(ECH, ECH)])

    return k(batch_question, e_space, word_emb, ent_emb)


NB = 8


def _tc_body(tt_ref, slen_ref, lastr_ref, batt_ref,
             qe_ref, wsa_ref, bsa_ref, watt_ref, rel_ref, ph_ref,
             w1_ref, b1_ref, w2_ref, b2_ref, er_ref, rsp_ref, am_ref,
             out_ref):
    f32 = jnp.float32
    g0 = pl.program_id(0)
    wsa = wsa_ref[...]
    bsa = bsa_ref[...]
    watt = watt_ref[...]
    rel = rel_ref[...]
    w1 = w1_ref[...]
    w2 = w2_ref[...]
    b1v = b1_ref[...]
    b2v = b2_ref[...]
    tt = tt_ref[0]
    aids = lax.broadcasted_iota(jnp.int32, (1, A), 1)
    sids = lax.broadcasted_iota(jnp.int32, (S, 1), 0)
    riota = lax.broadcasted_iota(jnp.int32, (R, A), 0)
    ones_col = jnp.ones((REL_DIM, 1), f32)
    ones_row = jnp.ones((1, R), f32)
    for i in range(NB):
        b = g0 * NB + i
        sv = jnp.tanh(
            jnp.dot(qe_ref[i], wsa, preferred_element_type=f32) + bsa)
        u = sv * watt
        logits = lax.dot_general(u, rel, (((1,), (1,)), ((), ())),
                                 preferred_element_type=f32) + batt_ref[0]
        logits = jnp.where(sids >= slen_ref[b], NEG, logits)
        m = jnp.max(logits, axis=0, keepdims=True)
        e = jnp.exp(logits - m)
        alpha = e / jnp.sum(e, axis=0, keepdims=True)
        raq = lax.dot_general(alpha, sv, (((0,), (0,)), ((), ())),
                              preferred_element_type=f32)
        base = jnp.dot(ph_ref[pl.ds(i, 1)], w1[:HIST_DIM],
                       preferred_element_type=f32)
        y = jnp.dot(raq, w1[HIST_DIM:], preferred_element_type=f32)
        z = jnp.maximum(base + y + b1v, 0.0)
        x2 = jnp.dot(z, w2, preferred_element_type=f32) + b2v
        s_rel = jnp.dot(x2[:, :REL_DIM] * rel, ones_col,
                        preferred_element_type=f32)
        p = lax.dot_general(x2[:, REL_DIM:], er_ref[i],
                            (((1,), (1,)), ((), ())),
                            preferred_element_type=f32)
        rsp = rsp_ref[pl.ds(i, 1)]
        oht = (rsp == riota).astype(f32)
        scores = jnp.dot(ones_row, oht * (p + s_rel),
                         preferred_element_type=f32)
        amask = am_ref[pl.ds(i, 1)]
        is_noop = (rsp == NO_OP).astype(jnp.int32)
        am_first = (1 - is_noop) * amask
        am_last = is_noop * amask
        jm = 1 - (lastr_ref[b] == NO_OP).astype(jnp.int32)
        selfl = (aids == 0).astype(jnp.int32)
        am_mid = jm * amask + (1 - jm) * selfl
        am = jnp.where(tt == 0, am_first,
                       jnp.where(tt == MAX_HOP - 1, am_last, am_mid))
        scores = jnp.where(am > 0, scores, NEG)
        mm = jnp.max(scores, axis=1, keepdims=True)
        ee = jnp.exp(scores - mm)
        out_ref[pl.ds(i, 1)] = ee / jnp.sum(ee, axis=1, keepdims=True)


def _tc_call_kwargs():
    smem = pl.BlockSpec(memory_space=pltpu.SMEM)
    full = pl.BlockSpec
    return dict(
        grid=(B // NB,),
        in_specs=[
            smem,
            smem,
            smem,
            smem,
            full((NB, S, WORD_DIM), lambda b: (b, 0, 0)),
            full((WORD_DIM, REL_DIM), lambda b: (0, 0)),
            full((1, REL_DIM), lambda b: (0, 0)),
            full((1, REL_DIM), lambda b: (0, 0)),
            full((R, REL_DIM), lambda b: (0, 0)),
            full((NB, HIST_DIM), lambda b: (b, 0)),
            full((HIST_DIM + REL_DIM, ACTION_DIM), lambda b: (0, 0)),
            full((1, ACTION_DIM), lambda b: (0, 0)),
            full((ACTION_DIM, ACTION_DIM), lambda b: (0, 0)),
            full((1, ACTION_DIM), lambda b: (0, 0)),
            full((NB, A, ENT_DIM), lambda b: (b, 0, 0)),
            full((NB, A), lambda b: (b, 0)),
            full((NB, A), lambda b: (b, 0)),
        ],
        out_specs=full((NB, A), lambda b: (b, 0)),
        out_shape=jax.ShapeDtypeStruct((B, A), jnp.float32),
        compiler_params=pltpu.CompilerParams(
            dimension_semantics=("arbitrary",)),
    )


def kernel(t, batch_question, batch_sent_len, batch_path_hidden, last_r,
           r_space, e_space, action_mask, word_emb, rel_emb, ent_emb,
           W_sa, b_sa, W_att, b_att, W1, b1, W2, b2):
    q_rows, e_rows = _sc_gather(batch_question, e_space, word_emb, ent_emb)

    tt = jnp.asarray(t, jnp.int32).reshape(1)
    wsa_t = jnp.take(W_sa, t, axis=0)
    bsa_t = jnp.take(b_sa, t, axis=0).reshape(1, REL_DIM)
    watt_row = W_att[:, 0].reshape(1, REL_DIM)

    return pl.pallas_call(_tc_body, **_tc_call_kwargs())(
        tt,
        batch_sent_len,
        last_r,
        b_att,
        q_rows,
        wsa_t,
        bsa_t,
        watt_row,
        rel_emb,
        batch_path_hidden,
        W1,
        b1.reshape(1, ACTION_DIM),
        W2,
        b2.reshape(1, ACTION_DIM),
        e_rows,
        r_space,
        action_mask,
    )


# speedup vs baseline: 374.7321x; 1.5077x over previous
import functools

import jax
import jax.numpy as jnp
from jax import lax
from jax.experimental import pallas as pl
from jax.experimental.pallas import tpu as pltpu
from jax.experimental.pallas import tpu_sc as plsc

B, S, R, A = 32, 64, 128, 256
WORD_DIM = 128
REL_DIM = 128
ENT_DIM = 128
HIST_DIM = 256
MAX_HOP = 3
ACTION_DIM = REL_DIM + ENT_DIM
NO_OP = 2
NEG = -1e9

ECH = 128


def _sc_gather(batch_question, e_space, word_emb, ent_emb):
    info = plsc.get_sparse_core_info()
    nc, ns = info.num_cores, info.num_subcores
    mesh = plsc.VectorSubcoreMesh(core_axis_name="c", subcore_axis_name="s")

    @functools.partial(
        pl.kernel,
        out_type=(
            jax.ShapeDtypeStruct((B, S, WORD_DIM), jnp.float32),
            jax.ShapeDtypeStruct((B, A, ENT_DIM), jnp.float32),
        ),
        mesh=mesh,
        scratch_types=[
            pltpu.VMEM((S,), jnp.int32),
            pltpu.VMEM((ECH,), jnp.int32),
            pltpu.VMEM((ECH,), jnp.int32),
            pltpu.VMEM((S, WORD_DIM), jnp.float32),
            pltpu.VMEM((ECH, ENT_DIM), jnp.float32),
            pltpu.VMEM((ECH, ENT_DIM), jnp.float32),
            pltpu.SemaphoreType.DMA,
            pltpu.SemaphoreType.DMA,
            pltpu.SemaphoreType.DMA,
        ],
    )
    def k(qidx_hbm, eidx_hbm, word_hbm, ent_hbm, qout_hbm, eout_hbm,
          qi_v, ei0_v, ei1_v, qr_v, er0_v, er1_v, sem_q, sem_e0, sem_e1):
        wid = lax.axis_index("s") * nc + lax.axis_index("c")
        pltpu.sync_copy(qidx_hbm.at[wid], qi_v)
        pltpu.sync_copy(eidx_hbm.at[wid, pl.ds(0, ECH)], ei0_v)
        pltpu.sync_copy(eidx_hbm.at[wid, pl.ds(ECH, ECH)], ei1_v)
        cq = pltpu.async_copy(word_hbm.at[qi_v], qr_v, sem_q)
        c0 = pltpu.async_copy(ent_hbm.at[ei0_v], er0_v, sem_e0)
        c1 = pltpu.async_copy(ent_hbm.at[ei1_v], er1_v, sem_e1)
        cq.wait()
        pltpu.sync_copy(qr_v, qout_hbm.at[wid])
        c0.wait()
        pltpu.sync_copy(er0_v, eout_hbm.at[wid, pl.ds(0, ECH)])
        c1.wait()
        pltpu.sync_copy(er1_v, eout_hbm.at[wid, pl.ds(ECH, ECH)])

    return k(batch_question, e_space, word_emb, ent_emb)


NB = 8


def _tc_body(tt_ref, slen_ref, lastr_ref, batt_ref,
             qe_ref, wsa_ref, bsa_ref, watt_ref, rel_ref, ph_ref,
             w1_ref, b1_ref, w2_ref, b2_ref, er_ref, rsp_ref, am_ref,
             out_ref):
    f32 = jnp.float32
    g0 = pl.program_id(0)
    wsa = wsa_ref[...]
    bsa = bsa_ref[...]
    watt = watt_ref[...]
    rel = rel_ref[...]
    w1 = w1_ref[...]
    w2 = w2_ref[...]
    b1v = b1_ref[...]
    b2v = b2_ref[...]
    tt = tt_ref[0]
    aids = lax.broadcasted_iota(jnp.int32, (1, A), 1)
    sids = lax.broadcasted_iota(jnp.int32, (S, 1), 0)
    riota = lax.broadcasted_iota(jnp.int32, (R, A), 0)
    ones_col = jnp.ones((REL_DIM, 1), f32)
    ones_row = jnp.ones((1, R), f32)
    qe_all = qe_ref[...].reshape(NB * S, WORD_DIM)
    sv_all = jnp.tanh(
        jnp.dot(qe_all, wsa, preferred_element_type=f32) + bsa)
    u_all = sv_all * watt
    l_all = lax.dot_general(u_all, rel, (((1,), (1,)), ((), ())),
                            preferred_element_type=f32) + batt_ref[0]
    raqs = []
    for i in range(NB):
        b = g0 * NB + i
        sv = sv_all[i * S:(i + 1) * S]
        logits = jnp.where(sids >= slen_ref[b], NEG,
                           l_all[i * S:(i + 1) * S])
        m = jnp.max(logits, axis=0, keepdims=True)
        e = jnp.exp(logits - m)
        alpha = e / jnp.sum(e, axis=0, keepdims=True)
        raqs.append(lax.dot_general(alpha, sv, (((0,), (0,)), ((), ())),
                                    preferred_element_type=f32))
    raq_all = jnp.concatenate(raqs, axis=0)
    base_all = jnp.dot(ph_ref[...], w1[:HIST_DIM],
                       preferred_element_type=f32)
    y_all = jnp.dot(raq_all, w1[HIST_DIM:], preferred_element_type=f32)
    zs = [jnp.maximum(y_all[i * R:(i + 1) * R] + base_all[i:i + 1]
                      + b1v, 0.0) for i in range(NB)]
    z_all = jnp.concatenate(zs, axis=0)
    x2_all = jnp.dot(z_all, w2, preferred_element_type=f32) + b2v
    for i in range(NB):
        b = g0 * NB + i
        x2 = x2_all[i * R:(i + 1) * R]
        s_rel = jnp.dot(x2[:, :REL_DIM] * rel, ones_col,
                        preferred_element_type=f32)
        p = lax.dot_general(x2[:, REL_DIM:], er_ref[i],
                            (((1,), (1,)), ((), ())),
                            preferred_element_type=f32)
        rsp = rsp_ref[pl.ds(i, 1)]
        oht = (rsp == riota).astype(f32)
        scores = jnp.dot(ones_row, oht * (p + s_rel),
                         preferred_element_type=f32)
        amask = am_ref[pl.ds(i, 1)]
        is_noop = (rsp == NO_OP).astype(jnp.int32)
        am_first = (1 - is_noop) * amask
        am_last = is_noop * amask
        jm = 1 - (lastr_ref[b] == NO_OP).astype(jnp.int32)
        selfl = (aids == 0).astype(jnp.int32)
        am_mid = jm * amask + (1 - jm) * selfl
        am = jnp.where(tt == 0, am_first,
                       jnp.where(tt == MAX_HOP - 1, am_last, am_mid))
        scores = jnp.where(am > 0, scores, NEG)
        mm = jnp.max(scores, axis=1, keepdims=True)
        ee = jnp.exp(scores - mm)
        out_ref[pl.ds(i, 1)] = ee / jnp.sum(ee, axis=1, keepdims=True)


def _tc_call_kwargs():
    smem = pl.BlockSpec(memory_space=pltpu.SMEM)
    full = pl.BlockSpec
    return dict(
        grid=(B // NB,),
        in_specs=[
            smem,
            smem,
            smem,
            smem,
            full((NB, S, WORD_DIM), lambda b: (b, 0, 0)),
            full((WORD_DIM, REL_DIM), lambda b: (0, 0)),
            full((1, REL_DIM), lambda b: (0, 0)),
            full((1, REL_DIM), lambda b: (0, 0)),
            full((R, REL_DIM), lambda b: (0, 0)),
            full((NB, HIST_DIM), lambda b: (b, 0)),
            full((HIST_DIM + REL_DIM, ACTION_DIM), lambda b: (0, 0)),
            full((1, ACTION_DIM), lambda b: (0, 0)),
            full((ACTION_DIM, ACTION_DIM), lambda b: (0, 0)),
            full((1, ACTION_DIM), lambda b: (0, 0)),
            full((NB, A, ENT_DIM), lambda b: (b, 0, 0)),
            full((NB, A), lambda b: (b, 0)),
            full((NB, A), lambda b: (b, 0)),
        ],
        out_specs=full((NB, A), lambda b: (b, 0)),
        out_shape=jax.ShapeDtypeStruct((B, A), jnp.float32),
        compiler_params=pltpu.CompilerParams(
            dimension_semantics=("arbitrary",)),
    )


def kernel(t, batch_question, batch_sent_len, batch_path_hidden, last_r,
           r_space, e_space, action_mask, word_emb, rel_emb, ent_emb,
           W_sa, b_sa, W_att, b_att, W1, b1, W2, b2):
    q_rows, e_rows = _sc_gather(batch_question, e_space, word_emb, ent_emb)

    tt = jnp.asarray(t, jnp.int32).reshape(1)
    wsa_t = jnp.take(W_sa, t, axis=0)
    bsa_t = jnp.take(b_sa, t, axis=0).reshape(1, REL_DIM)
    watt_row = W_att[:, 0].reshape(1, REL_DIM)

    return pl.pallas_call(_tc_body, **_tc_call_kwargs())(
        tt,
        batch_sent_len,
        last_r,
        b_att,
        q_rows,
        wsa_t,
        bsa_t,
        watt_row,
        rel_emb,
        batch_path_hidden,
        W1,
        b1.reshape(1, ACTION_DIM),
        W2,
        b2.reshape(1, ACTION_DIM),
        e_rows,
        r_space,
        action_mask,
    )


# speedup vs baseline: 379.7437x; 1.0134x over previous
import functools

import jax
import jax.numpy as jnp
from jax import lax
from jax.experimental import pallas as pl
from jax.experimental.pallas import tpu as pltpu
from jax.experimental.pallas import tpu_sc as plsc

B, S, R, A = 32, 64, 128, 256
WORD_DIM = 128
REL_DIM = 128
ENT_DIM = 128
HIST_DIM = 256
MAX_HOP = 3
ACTION_DIM = REL_DIM + ENT_DIM
NO_OP = 2
NEG = -1e9

ECH = 128


def _sc_gather(batch_question, e_space, word_emb, ent_emb):
    info = plsc.get_sparse_core_info()
    nc, ns = info.num_cores, info.num_subcores
    mesh = plsc.VectorSubcoreMesh(core_axis_name="c", subcore_axis_name="s")

    @functools.partial(
        pl.kernel,
        out_type=(
            jax.ShapeDtypeStruct((B, S, WORD_DIM), jnp.float32),
            jax.ShapeDtypeStruct((B, A, ENT_DIM), jnp.float32),
        ),
        mesh=mesh,
        scratch_types=[
            pltpu.VMEM((S,), jnp.int32),
            pltpu.VMEM((ECH,), jnp.int32),
            pltpu.VMEM((ECH,), jnp.int32),
            pltpu.VMEM((S, WORD_DIM), jnp.float32),
            pltpu.VMEM((ECH, ENT_DIM), jnp.float32),
            pltpu.VMEM((ECH, ENT_DIM), jnp.float32),
            pltpu.SemaphoreType.DMA,
            pltpu.SemaphoreType.DMA,
            pltpu.SemaphoreType.DMA,
        ],
    )
    def k(qidx_hbm, eidx_hbm, word_hbm, ent_hbm, qout_hbm, eout_hbm,
          qi_v, ei0_v, ei1_v, qr_v, er0_v, er1_v, sem_q, sem_e0, sem_e1):
        wid = lax.axis_index("s") * nc + lax.axis_index("c")
        pltpu.sync_copy(qidx_hbm.at[wid], qi_v)
        pltpu.sync_copy(eidx_hbm.at[wid, pl.ds(0, ECH)], ei0_v)
        pltpu.sync_copy(eidx_hbm.at[wid, pl.ds(ECH, ECH)], ei1_v)
        cq = pltpu.async_copy(word_hbm.at[qi_v], qr_v, sem_q)
        c0 = pltpu.async_copy(ent_hbm.at[ei0_v], er0_v, sem_e0)
        c1 = pltpu.async_copy(ent_hbm.at[ei1_v], er1_v, sem_e1)
        cq.wait()
        pltpu.sync_copy(qr_v, qout_hbm.at[wid])
        c0.wait()
        pltpu.sync_copy(er0_v, eout_hbm.at[wid, pl.ds(0, ECH)])
        c1.wait()
        pltpu.sync_copy(er1_v, eout_hbm.at[wid, pl.ds(ECH, ECH)])

    return k(batch_question, e_space, word_emb, ent_emb)


NB = 16


def _tc_body(tt_ref, slen_ref, lastr_ref, batt_ref,
             qe_ref, wsa_ref, bsa_ref, watt_ref, rel_ref, ph_ref,
             w1_ref, b1_ref, w2_ref, b2_ref, er_ref, rsp_ref, am_ref,
             out_ref):
    f32 = jnp.float32
    g0 = pl.program_id(0)
    wsa = wsa_ref[...]
    bsa = bsa_ref[...]
    watt = watt_ref[...]
    rel = rel_ref[...]
    w1 = w1_ref[...]
    w2 = w2_ref[...]
    b1v = b1_ref[...]
    b2v = b2_ref[...]
    tt = tt_ref[0]
    aids = lax.broadcasted_iota(jnp.int32, (1, A), 1)
    sids = lax.broadcasted_iota(jnp.int32, (S, 1), 0)
    riota = lax.broadcasted_iota(jnp.int32, (R, A), 0)
    ones_col = jnp.ones((REL_DIM, 1), f32)
    ones_row = jnp.ones((1, R), f32)
    qe_all = qe_ref[...].reshape(NB * S, WORD_DIM)
    sv_all = jnp.tanh(
        jnp.dot(qe_all, wsa, preferred_element_type=f32) + bsa)
    u_all = sv_all * watt
    l_all = lax.dot_general(u_all, rel, (((1,), (1,)), ((), ())),
                            preferred_element_type=f32) + batt_ref[0]
    raqs = []
    for i in range(NB):
        b = g0 * NB + i
        sv = sv_all[i * S:(i + 1) * S]
        logits = jnp.where(sids >= slen_ref[b], NEG,
                           l_all[i * S:(i + 1) * S])
        m = jnp.max(logits, axis=0, keepdims=True)
        e = jnp.exp(logits - m)
        alpha = e / jnp.sum(e, axis=0, keepdims=True)
        raqs.append(lax.dot_general(alpha, sv, (((0,), (0,)), ((), ())),
                                    preferred_element_type=f32))
    raq_all = jnp.concatenate(raqs, axis=0)
    base_all = jnp.dot(ph_ref[...], w1[:HIST_DIM],
                       preferred_element_type=f32)
    y_all = jnp.dot(raq_all, w1[HIST_DIM:], preferred_element_type=f32)
    zs = [jnp.maximum(y_all[i * R:(i + 1) * R] + base_all[i:i + 1]
                      + b1v, 0.0) for i in range(NB)]
    z_all = jnp.concatenate(zs, axis=0)
    x2_all = jnp.dot(z_all, w2, preferred_element_type=f32) + b2v
    for i in range(NB):
        b = g0 * NB + i
        x2 = x2_all[i * R:(i + 1) * R]
        s_rel = jnp.dot(x2[:, :REL_DIM] * rel, ones_col,
                        preferred_element_type=f32)
        p = lax.dot_general(x2[:, REL_DIM:], er_ref[i],
                            (((1,), (1,)), ((), ())),
                            preferred_element_type=f32)
        rsp = rsp_ref[pl.ds(i, 1)]
        oht = (rsp == riota).astype(f32)
        scores = jnp.dot(ones_row, oht * (p + s_rel),
                         preferred_element_type=f32)
        amask = am_ref[pl.ds(i, 1)]
        is_noop = (rsp == NO_OP).astype(jnp.int32)
        am_first = (1 - is_noop) * amask
        am_last = is_noop * amask
        jm = 1 - (lastr_ref[b] == NO_OP).astype(jnp.int32)
        selfl = (aids == 0).astype(jnp.int32)
        am_mid = jm * amask + (1 - jm) * selfl
        am = jnp.where(tt == 0, am_first,
                       jnp.where(tt == MAX_HOP - 1, am_last, am_mid))
        scores = jnp.where(am > 0, scores, NEG)
        mm = jnp.max(scores, axis=1, keepdims=True)
        ee = jnp.exp(scores - mm)
        out_ref[pl.ds(i, 1)] = ee / jnp.sum(ee, axis=1, keepdims=True)


def _tc_call_kwargs():
    smem = pl.BlockSpec(memory_space=pltpu.SMEM)
    full = pl.BlockSpec
    return dict(
        grid=(B // NB,),
        in_specs=[
            smem,
            smem,
            smem,
            smem,
            full((NB, S, WORD_DIM), lambda b: (b, 0, 0)),
            full((WORD_DIM, REL_DIM), lambda b: (0, 0)),
            full((1, REL_DIM), lambda b: (0, 0)),
            full((1, REL_DIM), lambda b: (0, 0)),
            full((R, REL_DIM), lambda b: (0, 0)),
            full((NB, HIST_DIM), lambda b: (b, 0)),
            full((HIST_DIM + REL_DIM, ACTION_DIM), lambda b: (0, 0)),
            full((1, ACTION_DIM), lambda b: (0, 0)),
            full((ACTION_DIM, ACTION_DIM), lambda b: (0, 0)),
            full((1, ACTION_DIM), lambda b: (0, 0)),
            full((NB, A, ENT_DIM), lambda b: (b, 0, 0)),
            full((NB, A), lambda b: (b, 0)),
            full((NB, A), lambda b: (b, 0)),
        ],
        out_specs=full((NB, A), lambda b: (b, 0)),
        out_shape=jax.ShapeDtypeStruct((B, A), jnp.float32),
        compiler_params=pltpu.CompilerParams(
            dimension_semantics=("arbitrary",)),
    )


def kernel(t, batch_question, batch_sent_len, batch_path_hidden, last_r,
           r_space, e_space, action_mask, word_emb, rel_emb, ent_emb,
           W_sa, b_sa, W_att, b_att, W1, b1, W2, b2):
    q_rows, e_rows = _sc_gather(batch_question, e_space, word_emb, ent_emb)

    tt = jnp.asarray(t, jnp.int32).reshape(1)
    wsa_t = jnp.take(W_sa, t, axis=0)
    bsa_t = jnp.take(b_sa, t, axis=0).reshape(1, REL_DIM)
    watt_row = W_att[:, 0].reshape(1, REL_DIM)

    return pl.pallas_call(_tc_body, **_tc_call_kwargs())(
        tt,
        batch_sent_len,
        last_r,
        b_att,
        q_rows,
        wsa_t,
        bsa_t,
        watt_row,
        rel_emb,
        batch_path_hidden,
        W1,
        b1.reshape(1, ACTION_DIM),
        W2,
        b2.reshape(1, ACTION_DIM),
        e_rows,
        r_space,
        action_mask,
    )
